# baseline (speedup 1.0000x reference)
import jax
import jax.numpy as jnp
from jax.experimental import pallas as pl
from jax.experimental.pallas import tpu as pltpu

_B, _DIM, _L = 65536, 256, 6
_BLK = 2048
_S = 0.1
_MAX_NORM = 10.0
_EPS = 1e-8
_NEPS2 = 1e-24


def _collapse_body(h_ref, wcat_ref, b1_ref, w2t_ref, b2_ref, anc_ref,
                   hout_ref, align_ref, div_ref, tens_ref):
    h = h_ref[...]
    wcat = wcat_ref[...].astype(jnp.bfloat16)
    w2t = w2t_ref[...].astype(jnp.bfloat16)
    b1 = b1_ref[...]
    b2 = b2_ref[...]
    anc = anc_ref[...]

    inv_an = jax.lax.rsqrt(
        jnp.maximum(jnp.sum(anc * anc, axis=1, keepdims=True), _NEPS2))
    anc_n = anc * inv_an
    inv_row = inv_an.reshape(1, 3)

    nrm2 = jnp.sum(h * h, axis=1, keepdims=True)

    for l in range(_L):
        mm1 = jax.lax.dot_general(
            h.astype(jnp.bfloat16), wcat, (((1,), (0,)), ((), ())),
            preferred_element_type=jnp.float32)
        z1 = mm1[:, :_DIM]
        raw_n = mm1[:, _DIM:_DIM + 3] * inv_row
        align = raw_n * jax.lax.rsqrt(jnp.maximum(nrm2, _NEPS2))
        div = 1.0 - align
        d2 = nrm2 - 2.0 * raw_n + 1.0
        c = div * jax.lax.rsqrt(jnp.maximum(d2, _NEPS2))
        a_scl = 1.0 - _S * jnp.sum(c, axis=1, keepdims=True)
        fsub = jax.lax.dot_general(
            c, anc_n, (((1,), (0,)), ((), ())),
            preferred_element_type=jnp.float32)

        t = jnp.tanh(z1 + b1).astype(jnp.bfloat16)
        delta = jax.lax.dot_general(
            t, w2t, (((1,), (0,)), ((), ())),
            preferred_element_type=jnp.float32) + b2

        h_new = h * a_scl + (delta + _S * fsub)
        n2 = jnp.sum(h_new * h_new, axis=1, keepdims=True)
        nrm = jnp.sqrt(n2)
        scale = jnp.where(nrm > _MAX_NORM, _MAX_NORM / (nrm + _EPS), 1.0)
        h = h_new * scale
        nrm2 = n2 * scale * scale

        align_t = align.T
        div_t = 1.0 - align_t
        align_ref[l] = align_t
        div_ref[l] = div_t
        tens_ref[l] = div_t * div_t

    hout_ref[...] = h


def kernel(h0, W1, b1, W2, b2, anchor_entail, anchor_contra, anchor_neutral):
    anc = jnp.stack([anchor_entail, anchor_contra, anchor_neutral])
    wcat = jnp.concatenate([W1.T, anc.T], axis=1)
    b1r = b1.reshape(1, _DIM)
    b2r = b2.reshape(1, _DIM)

    trace_shape = jax.ShapeDtypeStruct((_L, 3, _B), jnp.float32)
    trace_spec = pl.BlockSpec((_L, 3, _BLK), lambda i: (0, 0, i))

    h_final, al, dv, tn = pl.pallas_call(
        _collapse_body,
        grid=(_B // _BLK,),
        in_specs=[
            pl.BlockSpec((_BLK, _DIM), lambda i: (i, 0)),
            pl.BlockSpec((_DIM, _DIM + 3), lambda i: (0, 0)),
            pl.BlockSpec((1, _DIM), lambda i: (0, 0)),
            pl.BlockSpec((_DIM, _DIM), lambda i: (0, 0)),
            pl.BlockSpec((1, _DIM), lambda i: (0, 0)),
            pl.BlockSpec((3, _DIM), lambda i: (0, 0)),
        ],
        out_specs=[
            pl.BlockSpec((_BLK, _DIM), lambda i: (i, 0)),
            trace_spec, trace_spec, trace_spec,
        ],
        out_shape=[
            jax.ShapeDtypeStruct((_B, _DIM), jnp.float32),
            trace_shape, trace_shape, trace_shape,
        ],
        compiler_params=pltpu.CompilerParams(
            dimension_semantics=("parallel",)),
    )(h0, wcat, b1r, W2.T, b2r, anc)

    align_tr = jnp.transpose(al, (0, 2, 1))
    div_tr = jnp.transpose(dv, (0, 2, 1))
    tens_tr = jnp.transpose(tn, (0, 2, 1))
    return h_final, align_tr, div_tr, tens_tr


# speedup vs baseline: 1.0148x; 1.0148x over previous
import jax
import jax.numpy as jnp
from jax.experimental import pallas as pl
from jax.experimental.pallas import tpu as pltpu

_B, _DIM, _L = 65536, 256, 6
_BLK = 2048
_S = 0.1
_MAX_NORM = 10.0
_EPS = 1e-8
_NEPS2 = 1e-24


def _collapse_body(h_ref, wcat_ref, b1_ref, w2t_ref, b2_ref, anc_ref,
                   hout_ref, align_ref, div_ref, tens_ref):
    h = h_ref[...]
    wcat = wcat_ref[...]
    w2t = w2t_ref[...]
    b1 = b1_ref[...]
    b2 = b2_ref[...]
    anc = anc_ref[...]

    inv_an = jax.lax.rsqrt(
        jnp.maximum(jnp.sum(anc * anc, axis=1, keepdims=True), _NEPS2))
    anc_n = anc * inv_an
    inv_row = inv_an.reshape(1, 3)

    nrm2 = jnp.sum(h * h, axis=1, keepdims=True)

    for l in range(_L):
        mm1 = jax.lax.dot_general(
            h, wcat, (((1,), (0,)), ((), ())),
            preferred_element_type=jnp.float32)
        z1 = mm1[:, :_DIM]
        raw_n = mm1[:, _DIM:_DIM + 3] * inv_row
        align = raw_n * jax.lax.rsqrt(jnp.maximum(nrm2, _NEPS2))
        div = 1.0 - align
        d2 = nrm2 - 2.0 * raw_n + 1.0
        c = (_S * div) * jax.lax.rsqrt(jnp.maximum(d2, _NEPS2))
        a_scl = 1.0 - jnp.sum(c, axis=1, keepdims=True)
        fsub = jax.lax.dot_general(
            c, anc_n, (((1,), (0,)), ((), ())),
            preferred_element_type=jnp.float32)

        t = jnp.tanh(z1 + b1)
        delta = jax.lax.dot_general(
            t, w2t, (((1,), (0,)), ((), ())),
            preferred_element_type=jnp.float32) + b2

        h_new = h * a_scl + (delta + fsub)
        n2 = jnp.sum(h_new * h_new, axis=1, keepdims=True)
        nrm = jnp.sqrt(n2)
        scale = jnp.where(nrm > _MAX_NORM, _MAX_NORM / (nrm + _EPS), 1.0)
        h = h_new * scale
        nrm2 = n2 * scale * scale

        align_t = align.T
        div_t = 1.0 - align_t
        align_ref[l] = align_t
        div_ref[l] = div_t
        tens_ref[l] = div_t * div_t

    hout_ref[...] = h


def kernel(h0, W1, b1, W2, b2, anchor_entail, anchor_contra, anchor_neutral):
    anc = jnp.stack([anchor_entail, anchor_contra, anchor_neutral])
    wcat = jnp.concatenate([W1.T, anc.T], axis=1)
    b1r = b1.reshape(1, _DIM)
    b2r = b2.reshape(1, _DIM)

    trace_shape = jax.ShapeDtypeStruct((_L, 3, _B), jnp.float32)
    trace_spec = pl.BlockSpec((_L, 3, _BLK), lambda i: (0, 0, i))

    h_final, al, dv, tn = pl.pallas_call(
        _collapse_body,
        grid=(_B // _BLK,),
        in_specs=[
            pl.BlockSpec((_BLK, _DIM), lambda i: (i, 0)),
            pl.BlockSpec((_DIM, _DIM + 3), lambda i: (0, 0)),
            pl.BlockSpec((1, _DIM), lambda i: (0, 0)),
            pl.BlockSpec((_DIM, _DIM), lambda i: (0, 0)),
            pl.BlockSpec((1, _DIM), lambda i: (0, 0)),
            pl.BlockSpec((3, _DIM), lambda i: (0, 0)),
        ],
        out_specs=[
            pl.BlockSpec((_BLK, _DIM), lambda i: (i, 0)),
            trace_spec, trace_spec, trace_spec,
        ],
        out_shape=[
            jax.ShapeDtypeStruct((_B, _DIM), jnp.float32),
            trace_shape, trace_shape, trace_shape,
        ],
        compiler_params=pltpu.CompilerParams(
            dimension_semantics=("parallel",)),
    )(h0, wcat, b1r, W2.T, b2r, anc)

    align_tr = jnp.transpose(al, (0, 2, 1))
    div_tr = jnp.transpose(dv, (0, 2, 1))
    tens_tr = jnp.transpose(tn, (0, 2, 1))
    return h_final, align_tr, div_tr, tens_tr


# speedup vs baseline: 1.1716x; 1.1546x over previous
import jax
import jax.numpy as jnp
from jax.experimental import pallas as pl
from jax.experimental.pallas import tpu as pltpu

_B, _DIM, _L = 65536, 256, 6
_BLK = 2048
_S = 0.1
_MAX_NORM = 10.0
_EPS = 1e-8
_NEPS2 = 1e-24


def _collapse_body(h_ref, wcat_ref, b1_ref, w2t_ref, b2_ref, anc_ref,
                   hout_ref, align_ref, div_ref, tens_ref):
    h = h_ref[...]
    wcat = wcat_ref[...]
    w2t = w2t_ref[...]
    b1 = b1_ref[...]
    b2 = b2_ref[...]
    anc = anc_ref[...]

    inv_an = jax.lax.rsqrt(
        jnp.maximum(jnp.sum(anc * anc, axis=1, keepdims=True), _NEPS2))
    anc_n = anc * inv_an
    inv_row = inv_an.reshape(1, 3)

    nrm2 = jnp.sum(h * h, axis=1, keepdims=True)

    for l in range(_L):
        mm1 = jax.lax.dot_general(
            h, wcat, (((1,), (0,)), ((), ())),
            preferred_element_type=jnp.float32)
        z1 = mm1[:, :_DIM]
        raw_n = mm1[:, _DIM:_DIM + 3] * inv_row
        align = raw_n * jax.lax.rsqrt(jnp.maximum(nrm2, _NEPS2))
        div = 1.0 - align
        d2 = nrm2 - 2.0 * raw_n + 1.0
        c = div * jax.lax.rsqrt(jnp.maximum(d2, _NEPS2))
        a_scl = 1.0 - _S * jnp.sum(c, axis=1, keepdims=True)
        fsub = jax.lax.dot_general(
            c, anc_n, (((1,), (0,)), ((), ())),
            preferred_element_type=jnp.float32)

        t = jnp.tanh(z1 + b1)
        delta = jax.lax.dot_general(
            t, w2t, (((1,), (0,)), ((), ())),
            preferred_element_type=jnp.float32) + b2

        h_new = h * a_scl + (delta + _S * fsub)
        n2 = jnp.sum(h_new * h_new, axis=1, keepdims=True)
        scale = jnp.minimum(
            _MAX_NORM * jax.lax.rsqrt(jnp.maximum(n2, _NEPS2)), 1.0)
        h = h_new * scale
        nrm2 = n2 * scale * scale

        align_t = align.T
        div_t = 1.0 - align_t
        align_ref[l] = align_t
        div_ref[l] = div_t
        tens_ref[l] = div_t * div_t

    hout_ref[...] = h


def kernel(h0, W1, b1, W2, b2, anchor_entail, anchor_contra, anchor_neutral):
    anc = jnp.stack([anchor_entail, anchor_contra, anchor_neutral])
    wcat = jnp.concatenate([W1.T, anc.T], axis=1)
    b1r = b1.reshape(1, _DIM)
    b2r = b2.reshape(1, _DIM)

    trace_shape = jax.ShapeDtypeStruct((_L, 3, _B), jnp.float32)
    trace_spec = pl.BlockSpec((_L, 3, _BLK), lambda i: (0, 0, i))

    h_final, al, dv, tn = pl.pallas_call(
        _collapse_body,
        grid=(_B // _BLK,),
        in_specs=[
            pl.BlockSpec((_BLK, _DIM), lambda i: (i, 0)),
            pl.BlockSpec((_DIM, _DIM + 3), lambda i: (0, 0)),
            pl.BlockSpec((1, _DIM), lambda i: (0, 0)),
            pl.BlockSpec((_DIM, _DIM), lambda i: (0, 0)),
            pl.BlockSpec((1, _DIM), lambda i: (0, 0)),
            pl.BlockSpec((3, _DIM), lambda i: (0, 0)),
        ],
        out_specs=[
            pl.BlockSpec((_BLK, _DIM), lambda i: (i, 0)),
            trace_spec, trace_spec, trace_spec,
        ],
        out_shape=[
            jax.ShapeDtypeStruct((_B, _DIM), jnp.float32),
            trace_shape, trace_shape, trace_shape,
        ],
        compiler_params=pltpu.CompilerParams(
            dimension_semantics=("parallel",)),
    )(h0, wcat, b1r, W2.T, b2r, anc)

    align_tr = jnp.transpose(al, (0, 2, 1))
    div_tr = jnp.transpose(dv, (0, 2, 1))
    tens_tr = jnp.transpose(tn, (0, 2, 1))
    return h_final, align_tr, div_tr, tens_tr
